# Initial kernel scaffold; baseline (speedup 1.0000x reference)
import jax
import jax.numpy as jnp
from jax.experimental import pallas as pl


def kernel(z_e, codebook):
    raise NotImplementedError("write your pallas kernel here")


# speedup vs baseline: 1.1248x; 1.1248x over previous
import functools

import jax
import jax.numpy as jnp
from jax import lax
from jax.experimental import pallas as pl
from jax.experimental.pallas import tpu as pltpu
from jax.experimental.pallas import tpu_sc as plsc

N_TOKENS = 16384
N_CODES = 8192
DIM = 32
T_TILE = 256
C_CHUNK = 2048
S_BLOCK = 4096
N_WORKERS = 32
ROWS_PER_W = N_TOKENS // N_WORKERS
IDX_CHUNK = 128


_DN = (((1,), (1,)), ((), ()))


def _argmin_body(x_ref, cb_ref, cn_ref, xn_ref, idx_ref, lsum_ref):
    x = x_ref[...]
    xn = xn_ref[...]
    macc = jnp.full((T_TILE, 1), jnp.inf, dtype=jnp.float32)
    msel = jnp.full((T_TILE, 1), jnp.inf, dtype=jnp.float32)
    midx = jnp.zeros((T_TILE, 1), dtype=jnp.int32)
    inner = S_BLOCK // C_CHUNK
    for b in range(N_CODES // S_BLOCK):
        bmin = jnp.full((T_TILE, 1), jnp.inf, dtype=jnp.float32)
        bidx = jnp.zeros((T_TILE, 1), dtype=jnp.int32)
        for k in range(inner):
            c = b * inner + k
            cb = cb_ref[c * C_CHUNK:(c + 1) * C_CHUNK, :]
            cn = cn_ref[:, c * C_CHUNK:(c + 1) * C_CHUNK]
            dot = lax.dot_general(x, cb, _DN,
                                  preferred_element_type=jnp.float32)
            s = (xn - 2.0 * dot) + cn
            cm = jnp.min(s, axis=1, keepdims=True)
            iota = lax.broadcasted_iota(jnp.int32, s.shape, 1) + c * C_CHUNK
            ci = jnp.min(jnp.where(s == cm, iota, jnp.int32(2 ** 30)),
                         axis=1, keepdims=True)
            updk = cm < bmin
            bidx = jnp.where(updk, ci, bidx)
            bmin = jnp.where(updk, cm, bmin)
        upd = bmin < macc
        midx = jnp.where(upd, bidx, midx)
        msel = jnp.where(upd, bmin, msel)
        bq = bmin.astype(jnp.bfloat16).astype(jnp.float32)
        macc = jnp.where(upd, bq, macc)
    idx_ref[...] = midx

    @pl.when(pl.program_id(0) == 0)
    def _():
        lsum_ref[...] = jnp.zeros_like(lsum_ref)

    lsum_ref[...] += jnp.sum(msel).reshape(1, 1)


def _argmin_call(xb, codebook, cn, xn):
    grid = (N_TOKENS // T_TILE,)
    return pl.pallas_call(
        _argmin_body,
        grid=grid,
        in_specs=[
            pl.BlockSpec((T_TILE, DIM), lambda i: (i, 0)),
            pl.BlockSpec((N_CODES, DIM), lambda i: (0, 0)),
            pl.BlockSpec((1, N_CODES), lambda i: (0, 0)),
            pl.BlockSpec((T_TILE, 1), lambda i: (i, 0)),
        ],
        out_specs=[
            pl.BlockSpec((T_TILE, 1), lambda i: (i, 0)),
            pl.BlockSpec((1, 1), lambda i: (0, 0)),
        ],
        out_shape=[
            jax.ShapeDtypeStruct((N_TOKENS, 1), jnp.int32),
            jax.ShapeDtypeStruct((1, 1), jnp.float32),
        ],
    )(xb, codebook, cn, xn)


D_PAD = 128


@functools.cache
def _sc_gather_kernel():
    mesh = plsc.VectorSubcoreMesh(core_axis_name="c", subcore_axis_name="s")

    @functools.partial(
        pl.kernel,
        mesh=mesh,
        out_type=jax.ShapeDtypeStruct((N_TOKENS, D_PAD), jnp.float32),
        scratch_types=[
            pltpu.VMEM((ROWS_PER_W,), jnp.int32),
            pltpu.VMEM((ROWS_PER_W, D_PAD), jnp.float32),
            pltpu.SemaphoreType.DMA,
        ],
    )
    def _sc_gather(table_hbm, idx_hbm, out_hbm, idx_v, rows_v, sem):
        n_chunks = ROWS_PER_W // IDX_CHUNK
        wid = lax.axis_index("s") * 2 + lax.axis_index("c")
        pltpu.sync_copy(idx_hbm.at[pl.ds(wid * ROWS_PER_W, ROWS_PER_W)], idx_v)
        copies = []
        for j in range(n_chunks):
            copies.append(pltpu.async_copy(
                table_hbm.at[idx_v.at[pl.ds(j * IDX_CHUNK, IDX_CHUNK)]],
                rows_v.at[pl.ds(j * IDX_CHUNK, IDX_CHUNK)],
                sem,
            ))
        for cp in copies:
            cp.wait()
        pltpu.sync_copy(rows_v, out_hbm.at[pl.ds(wid * ROWS_PER_W, ROWS_PER_W)])

    return _sc_gather


def kernel(z_e, codebook):
    B, L, D = z_e.shape
    flat = z_e.reshape(-1, D)
    xb = flat.astype(jnp.bfloat16)
    xn = jnp.sum(flat * flat, axis=1, keepdims=True)
    cn = jnp.sum(codebook * codebook, axis=1).reshape(1, N_CODES)

    idx2, lsum = _argmin_call(xb, codebook, cn, xn)
    idx = idx2.reshape(N_TOKENS)

    table_pad = jnp.pad(codebook, ((0, 0), (0, D_PAD - DIM)))
    zq_pad = _sc_gather_kernel()(table_pad, idx)
    z_q = zq_pad[:, :DIM].reshape(B, L, D)

    m = lsum[0, 0] / jnp.float32(N_TOKENS * DIM)
    loss = m + 0.25 * m
    z_q_st = z_e + (z_q - z_e)
    return (z_q_st, idx.reshape(B, L), loss)


# speedup vs baseline: 1.2363x; 1.0991x over previous
import functools

import jax
import jax.numpy as jnp
from jax import lax
from jax.experimental import pallas as pl
from jax.experimental.pallas import tpu as pltpu
from jax.experimental.pallas import tpu_sc as plsc

N_TOKENS = 16384
N_CODES = 8192
DIM = 32
T_TILE = 512
C_CHUNK = 2048
S_BLOCK = 4096
N_WORKERS = 32
ROWS_PER_W = N_TOKENS // N_WORKERS
IDX_CHUNK = 128


_DN = (((1,), (1,)), ((), ()))


def _argmin_body(x_ref, cb_ref, cn_ref, xn_ref, idx_ref, lsum_ref):
    x = x_ref[...]
    xn = xn_ref[...]
    macc = jnp.full((T_TILE, 1), jnp.inf, dtype=jnp.float32)
    msel = jnp.full((T_TILE, 1), jnp.inf, dtype=jnp.float32)
    midx = jnp.zeros((T_TILE, 1), dtype=jnp.int32)
    inner = S_BLOCK // C_CHUNK
    for b in range(N_CODES // S_BLOCK):
        bmin = jnp.full((T_TILE, 1), jnp.inf, dtype=jnp.float32)
        bidx = jnp.zeros((T_TILE, 1), dtype=jnp.int32)
        for k in range(inner):
            c = b * inner + k
            cb = cb_ref[c * C_CHUNK:(c + 1) * C_CHUNK, :]
            cn = cn_ref[:, c * C_CHUNK:(c + 1) * C_CHUNK]
            dot = lax.dot_general(x, cb, _DN,
                                  preferred_element_type=jnp.float32)
            s = (xn - 2.0 * dot) + cn
            cm = jnp.min(s, axis=1, keepdims=True)
            iota = lax.broadcasted_iota(jnp.int32, s.shape, 1) + c * C_CHUNK
            ci = jnp.min(jnp.where(s == cm, iota, jnp.int32(2 ** 30)),
                         axis=1, keepdims=True)
            updk = cm < bmin
            bidx = jnp.where(updk, ci, bidx)
            bmin = jnp.where(updk, cm, bmin)
        upd = bmin < macc
        midx = jnp.where(upd, bidx, midx)
        msel = jnp.where(upd, bmin, msel)
        bq = bmin.astype(jnp.bfloat16).astype(jnp.float32)
        macc = jnp.where(upd, bq, macc)
    idx_ref[...] = midx

    @pl.when(pl.program_id(0) == 0)
    def _():
        lsum_ref[...] = jnp.zeros_like(lsum_ref)

    lsum_ref[...] += jnp.sum(msel).reshape(1, 1)


def _argmin_call(xb, codebook, cn, xn):
    grid = (N_TOKENS // T_TILE,)
    return pl.pallas_call(
        _argmin_body,
        grid=grid,
        in_specs=[
            pl.BlockSpec((T_TILE, DIM), lambda i: (i, 0)),
            pl.BlockSpec((N_CODES, DIM), lambda i: (0, 0)),
            pl.BlockSpec((1, N_CODES), lambda i: (0, 0)),
            pl.BlockSpec((T_TILE, 1), lambda i: (i, 0)),
        ],
        out_specs=[
            pl.BlockSpec((T_TILE, 1), lambda i: (i, 0)),
            pl.BlockSpec((1, 1), lambda i: (0, 0)),
        ],
        out_shape=[
            jax.ShapeDtypeStruct((N_TOKENS, 1), jnp.int32),
            jax.ShapeDtypeStruct((1, 1), jnp.float32),
        ],
    )(xb, codebook, cn, xn)


D_PAD = 128


@functools.cache
def _sc_gather_kernel():
    mesh = plsc.VectorSubcoreMesh(core_axis_name="c", subcore_axis_name="s")

    @functools.partial(
        pl.kernel,
        mesh=mesh,
        out_type=jax.ShapeDtypeStruct((N_TOKENS, D_PAD), jnp.float32),
        scratch_types=[
            pltpu.VMEM((ROWS_PER_W,), jnp.int32),
            pltpu.VMEM((ROWS_PER_W, D_PAD), jnp.float32),
            pltpu.SemaphoreType.DMA,
        ],
    )
    def _sc_gather(table_hbm, idx_hbm, out_hbm, idx_v, rows_v, sem):
        n_chunks = ROWS_PER_W // IDX_CHUNK
        wid = lax.axis_index("s") * 2 + lax.axis_index("c")
        pltpu.sync_copy(idx_hbm.at[pl.ds(wid * ROWS_PER_W, ROWS_PER_W)], idx_v)
        copies = []
        for j in range(n_chunks):
            copies.append(pltpu.async_copy(
                table_hbm.at[idx_v.at[pl.ds(j * IDX_CHUNK, IDX_CHUNK)]],
                rows_v.at[pl.ds(j * IDX_CHUNK, IDX_CHUNK)],
                sem,
            ))
        for cp in copies:
            cp.wait()
        pltpu.sync_copy(rows_v, out_hbm.at[pl.ds(wid * ROWS_PER_W, ROWS_PER_W)])

    return _sc_gather


def kernel(z_e, codebook):
    B, L, D = z_e.shape
    flat = z_e.reshape(-1, D)
    xb = flat.astype(jnp.bfloat16)
    xn = jnp.sum(flat * flat, axis=1, keepdims=True)
    cn = jnp.sum(codebook * codebook, axis=1).reshape(1, N_CODES)

    idx2, lsum = _argmin_call(xb, codebook, cn, xn)
    idx = idx2.reshape(N_TOKENS)

    table_pad = jnp.pad(codebook, ((0, 0), (0, D_PAD - DIM)))
    zq_pad = _sc_gather_kernel()(table_pad, idx)
    z_q = zq_pad[:, :DIM].reshape(B, L, D)

    m = lsum[0, 0] / jnp.float32(N_TOKENS * DIM)
    loss = m + 0.25 * m
    z_q_st = z_e + (z_q - z_e)
    return (z_q_st, idx.reshape(B, L), loss)


# speedup vs baseline: 1.2944x; 1.0469x over previous
import functools

import jax
import jax.numpy as jnp
from jax import lax
from jax.experimental import pallas as pl
from jax.experimental.pallas import tpu as pltpu
from jax.experimental.pallas import tpu_sc as plsc

N_TOKENS = 16384
N_CODES = 8192
DIM = 32
T_TILE = 1024
C_CHUNK = 2048
S_BLOCK = 4096
N_WORKERS = 32
ROWS_PER_W = N_TOKENS // N_WORKERS
IDX_CHUNK = 128


_DN = (((1,), (1,)), ((), ()))


def _argmin_body(x_ref, cb_ref, cn_ref, xn_ref, idx_ref, lsum_ref):
    x = x_ref[...]
    xn = xn_ref[...]
    macc = jnp.full((T_TILE, 1), jnp.inf, dtype=jnp.float32)
    msel = jnp.full((T_TILE, 1), jnp.inf, dtype=jnp.float32)
    midx = jnp.zeros((T_TILE, 1), dtype=jnp.int32)
    inner = S_BLOCK // C_CHUNK
    for b in range(N_CODES // S_BLOCK):
        bmin = jnp.full((T_TILE, 1), jnp.inf, dtype=jnp.float32)
        bidx = jnp.zeros((T_TILE, 1), dtype=jnp.int32)
        for k in range(inner):
            c = b * inner + k
            cb = cb_ref[c * C_CHUNK:(c + 1) * C_CHUNK, :]
            cn = cn_ref[:, c * C_CHUNK:(c + 1) * C_CHUNK]
            dot = lax.dot_general(x, cb, _DN,
                                  preferred_element_type=jnp.float32)
            s = (xn - 2.0 * dot) + cn
            cm = jnp.min(s, axis=1, keepdims=True)
            iota = lax.broadcasted_iota(jnp.int32, s.shape, 1) + c * C_CHUNK
            ci = jnp.min(jnp.where(s == cm, iota, jnp.int32(2 ** 30)),
                         axis=1, keepdims=True)
            updk = cm < bmin
            bidx = jnp.where(updk, ci, bidx)
            bmin = jnp.where(updk, cm, bmin)
        upd = bmin < macc
        midx = jnp.where(upd, bidx, midx)
        msel = jnp.where(upd, bmin, msel)
        bq = bmin.astype(jnp.bfloat16).astype(jnp.float32)
        macc = jnp.where(upd, bq, macc)
    idx_ref[...] = midx

    @pl.when(pl.program_id(0) == 0)
    def _():
        lsum_ref[...] = jnp.zeros_like(lsum_ref)

    lsum_ref[...] += jnp.sum(msel).reshape(1, 1)


def _argmin_call(xb, codebook, cn, xn):
    grid = (N_TOKENS // T_TILE,)
    return pl.pallas_call(
        _argmin_body,
        grid=grid,
        in_specs=[
            pl.BlockSpec((T_TILE, DIM), lambda i: (i, 0)),
            pl.BlockSpec((N_CODES, DIM), lambda i: (0, 0)),
            pl.BlockSpec((1, N_CODES), lambda i: (0, 0)),
            pl.BlockSpec((T_TILE, 1), lambda i: (i, 0)),
        ],
        out_specs=[
            pl.BlockSpec((T_TILE, 1), lambda i: (i, 0)),
            pl.BlockSpec((1, 1), lambda i: (0, 0)),
        ],
        out_shape=[
            jax.ShapeDtypeStruct((N_TOKENS, 1), jnp.int32),
            jax.ShapeDtypeStruct((1, 1), jnp.float32),
        ],
    )(xb, codebook, cn, xn)


D_PAD = 128


@functools.cache
def _sc_gather_kernel():
    mesh = plsc.VectorSubcoreMesh(core_axis_name="c", subcore_axis_name="s")

    @functools.partial(
        pl.kernel,
        mesh=mesh,
        out_type=jax.ShapeDtypeStruct((N_TOKENS, D_PAD), jnp.float32),
        scratch_types=[
            pltpu.VMEM((ROWS_PER_W,), jnp.int32),
            pltpu.VMEM((ROWS_PER_W, D_PAD), jnp.float32),
            pltpu.SemaphoreType.DMA,
        ],
    )
    def _sc_gather(table_hbm, idx_hbm, out_hbm, idx_v, rows_v, sem):
        n_chunks = ROWS_PER_W // IDX_CHUNK
        wid = lax.axis_index("s") * 2 + lax.axis_index("c")
        pltpu.sync_copy(idx_hbm.at[pl.ds(wid * ROWS_PER_W, ROWS_PER_W)], idx_v)
        copies = []
        for j in range(n_chunks):
            copies.append(pltpu.async_copy(
                table_hbm.at[idx_v.at[pl.ds(j * IDX_CHUNK, IDX_CHUNK)]],
                rows_v.at[pl.ds(j * IDX_CHUNK, IDX_CHUNK)],
                sem,
            ))
        for cp in copies:
            cp.wait()
        pltpu.sync_copy(rows_v, out_hbm.at[pl.ds(wid * ROWS_PER_W, ROWS_PER_W)])

    return _sc_gather


def kernel(z_e, codebook):
    B, L, D = z_e.shape
    flat = z_e.reshape(-1, D)
    xb = flat.astype(jnp.bfloat16)
    xn = jnp.sum(flat * flat, axis=1, keepdims=True)
    cn = jnp.sum(codebook * codebook, axis=1).reshape(1, N_CODES)

    idx2, lsum = _argmin_call(xb, codebook, cn, xn)
    idx = idx2.reshape(N_TOKENS)

    table_pad = jnp.pad(codebook, ((0, 0), (0, D_PAD - DIM)))
    zq_pad = _sc_gather_kernel()(table_pad, idx)
    z_q = zq_pad[:, :DIM].reshape(B, L, D)

    m = lsum[0, 0] / jnp.float32(N_TOKENS * DIM)
    loss = m + 0.25 * m
    z_q_st = z_e + (z_q - z_e)
    return (z_q_st, idx.reshape(B, L), loss)


# speedup vs baseline: 1.3325x; 1.0295x over previous
import functools

import jax
import jax.numpy as jnp
from jax import lax
from jax.experimental import pallas as pl
from jax.experimental.pallas import tpu as pltpu
from jax.experimental.pallas import tpu_sc as plsc

N_TOKENS = 16384
N_CODES = 8192
DIM = 32
T_TILE = 2048
C_CHUNK = 2048
S_BLOCK = 4096
N_WORKERS = 32
ROWS_PER_W = N_TOKENS // N_WORKERS
IDX_CHUNK = 128


_DN = (((1,), (1,)), ((), ()))


def _argmin_body(x_ref, cb_ref, cn_ref, xn_ref, idx_ref, lsum_ref):
    x = x_ref[...]
    xn = xn_ref[...]
    macc = jnp.full((T_TILE, 1), jnp.inf, dtype=jnp.float32)
    msel = jnp.full((T_TILE, 1), jnp.inf, dtype=jnp.float32)
    midx = jnp.zeros((T_TILE, 1), dtype=jnp.int32)
    inner = S_BLOCK // C_CHUNK
    for b in range(N_CODES // S_BLOCK):
        bmin = jnp.full((T_TILE, 1), jnp.inf, dtype=jnp.float32)
        bidx = jnp.zeros((T_TILE, 1), dtype=jnp.int32)
        for k in range(inner):
            c = b * inner + k
            cb = cb_ref[c * C_CHUNK:(c + 1) * C_CHUNK, :]
            cn = cn_ref[:, c * C_CHUNK:(c + 1) * C_CHUNK]
            dot = lax.dot_general(x, cb, _DN,
                                  preferred_element_type=jnp.float32)
            s = (xn - 2.0 * dot) + cn
            cm = jnp.min(s, axis=1, keepdims=True)
            iota = lax.broadcasted_iota(jnp.int32, s.shape, 1) + c * C_CHUNK
            ci = jnp.min(jnp.where(s == cm, iota, jnp.int32(2 ** 30)),
                         axis=1, keepdims=True)
            updk = cm < bmin
            bidx = jnp.where(updk, ci, bidx)
            bmin = jnp.where(updk, cm, bmin)
        upd = bmin < macc
        midx = jnp.where(upd, bidx, midx)
        msel = jnp.where(upd, bmin, msel)
        bq = bmin.astype(jnp.bfloat16).astype(jnp.float32)
        macc = jnp.where(upd, bq, macc)
    idx_ref[...] = midx

    @pl.when(pl.program_id(0) == 0)
    def _():
        lsum_ref[...] = jnp.zeros_like(lsum_ref)

    lsum_ref[...] += jnp.sum(msel).reshape(1, 1)


def _argmin_call(xb, codebook, cn, xn):
    grid = (N_TOKENS // T_TILE,)
    return pl.pallas_call(
        _argmin_body,
        grid=grid,
        in_specs=[
            pl.BlockSpec((T_TILE, DIM), lambda i: (i, 0)),
            pl.BlockSpec((N_CODES, DIM), lambda i: (0, 0)),
            pl.BlockSpec((1, N_CODES), lambda i: (0, 0)),
            pl.BlockSpec((T_TILE, 1), lambda i: (i, 0)),
        ],
        out_specs=[
            pl.BlockSpec((T_TILE, 1), lambda i: (i, 0)),
            pl.BlockSpec((1, 1), lambda i: (0, 0)),
        ],
        out_shape=[
            jax.ShapeDtypeStruct((N_TOKENS, 1), jnp.int32),
            jax.ShapeDtypeStruct((1, 1), jnp.float32),
        ],
    )(xb, codebook, cn, xn)


D_PAD = 128


@functools.cache
def _sc_gather_kernel():
    mesh = plsc.VectorSubcoreMesh(core_axis_name="c", subcore_axis_name="s")

    @functools.partial(
        pl.kernel,
        mesh=mesh,
        out_type=jax.ShapeDtypeStruct((N_TOKENS, D_PAD), jnp.float32),
        scratch_types=[
            pltpu.VMEM((ROWS_PER_W,), jnp.int32),
            pltpu.VMEM((ROWS_PER_W, D_PAD), jnp.float32),
            pltpu.SemaphoreType.DMA,
        ],
    )
    def _sc_gather(table_hbm, idx_hbm, out_hbm, idx_v, rows_v, sem):
        n_chunks = ROWS_PER_W // IDX_CHUNK
        wid = lax.axis_index("s") * 2 + lax.axis_index("c")
        pltpu.sync_copy(idx_hbm.at[pl.ds(wid * ROWS_PER_W, ROWS_PER_W)], idx_v)
        copies = []
        for j in range(n_chunks):
            copies.append(pltpu.async_copy(
                table_hbm.at[idx_v.at[pl.ds(j * IDX_CHUNK, IDX_CHUNK)]],
                rows_v.at[pl.ds(j * IDX_CHUNK, IDX_CHUNK)],
                sem,
            ))
        for cp in copies:
            cp.wait()
        pltpu.sync_copy(rows_v, out_hbm.at[pl.ds(wid * ROWS_PER_W, ROWS_PER_W)])

    return _sc_gather


def kernel(z_e, codebook):
    B, L, D = z_e.shape
    flat = z_e.reshape(-1, D)
    xb = flat.astype(jnp.bfloat16)
    xn = jnp.sum(flat * flat, axis=1, keepdims=True)
    cn = jnp.sum(codebook * codebook, axis=1).reshape(1, N_CODES)

    idx2, lsum = _argmin_call(xb, codebook, cn, xn)
    idx = idx2.reshape(N_TOKENS)

    table_pad = jnp.pad(codebook, ((0, 0), (0, D_PAD - DIM)))
    zq_pad = _sc_gather_kernel()(table_pad, idx)
    z_q = zq_pad[:, :DIM].reshape(B, L, D)

    m = lsum[0, 0] / jnp.float32(N_TOKENS * DIM)
    loss = m + 0.25 * m
    z_q_st = z_e + (z_q - z_e)
    return (z_q_st, idx.reshape(B, L), loss)
